# Initial kernel scaffold; baseline (speedup 1.0000x reference)
import jax
import jax.numpy as jnp
from jax.experimental import pallas as pl


def kernel(node_features_list_0, node_features_list_1, node_features_list_2, w0, aw0, w1, aw1, fc_w, fc_b):
    raise NotImplementedError("write your pallas kernel here")


# speedup vs baseline: 3.2258x; 3.2258x over previous
import jax
import jax.numpy as jnp
from jax.experimental import pallas as pl
from jax.experimental.pallas import tpu as pltpu

_D = 128
_N = 16
_BATCH = 2048
_S = 64


def _fused(h0_ref, h1_ref, h2_ref, w0_ref, aw0_ref, w1_ref, aw1_ref,
           fcw_ref, fcb_ref, out_ref):
    f32 = jnp.float32
    m2 = jnp.mean(h2_ref[...], axis=1)
    h1 = h1_ref[...]
    h1f = h1.reshape(_S * _N, _D)
    a1 = jnp.maximum(
        jnp.dot(h1f, w0_ref[...], preferred_element_type=f32)
        + jnp.dot(m2, aw0_ref[...], preferred_element_type=f32), 0.0)
    m1 = jnp.mean(h1, axis=1)
    h0 = h0_ref[...]
    a0 = jnp.maximum(
        jnp.dot(h0, w0_ref[...], preferred_element_type=f32)
        + jnp.dot(m1, aw0_ref[...], preferred_element_type=f32), 0.0)
    ma1 = jnp.mean(a1.reshape(_S, _N, _D), axis=1)
    w1 = w1_ref[...]
    aw1 = aw1_ref[...]
    hid = (jnp.dot(a0, w1[:_D], preferred_element_type=f32)
           + jnp.dot(h0, w1[_D:], preferred_element_type=f32)
           + jnp.dot(ma1, aw1[:_D], preferred_element_type=f32)
           + jnp.dot(m1, aw1[_D:], preferred_element_type=f32))
    fcw = fcw_ref[...]
    out_ref[...] = (jnp.dot(hid, fcw[:_D], preferred_element_type=f32)
                    + jnp.dot(h0, fcw[_D:], preferred_element_type=f32)
                    + fcb_ref[0])


def kernel(node_features_list_0, node_features_list_1, node_features_list_2,
           w0, aw0, w1, aw1, fc_w, fc_b):
    h0 = node_features_list_0
    h1 = node_features_list_1.reshape(_BATCH, _N, _D)
    h2 = node_features_list_2.reshape(_BATCH * _N, _N, _D)
    grid = (_BATCH // _S,)
    rep2 = lambda i: (0, 0)
    return pl.pallas_call(
        _fused,
        grid=grid,
        in_specs=[
            pl.BlockSpec((_S, _D), lambda i: (i, 0)),
            pl.BlockSpec((_S, _N, _D), lambda i: (i, 0, 0)),
            pl.BlockSpec((_S * _N, _N, _D), lambda i: (i, 0, 0)),
            pl.BlockSpec((_D, _D), rep2),
            pl.BlockSpec((_D, _D), rep2),
            pl.BlockSpec((2 * _D, _D), rep2),
            pl.BlockSpec((2 * _D, _D), rep2),
            pl.BlockSpec((2 * _D, 1), rep2),
            pl.BlockSpec(memory_space=pltpu.SMEM),
        ],
        out_specs=pl.BlockSpec((_S, 1), lambda i: (i, 0)),
        out_shape=jax.ShapeDtypeStruct((_BATCH, 1), jnp.float32),
        compiler_params=pltpu.CompilerParams(
            dimension_semantics=("arbitrary",),
        ),
    )(h0, h1, h2, w0, aw0, w1, aw1, fc_w, fc_b)


# speedup vs baseline: 3.3138x; 1.0273x over previous
import jax
import jax.numpy as jnp
from jax.experimental import pallas as pl
from jax.experimental.pallas import tpu as pltpu

_D = 128
_N = 16
_BATCH = 2048
_S = 128


def _fused(h0_ref, h1_ref, h2_ref, w0_ref, aw0_ref, w1_ref, aw1_ref,
           fcw_ref, fcb_ref, out_ref):
    f32 = jnp.float32
    m2 = jnp.mean(h2_ref[...], axis=1)
    h1 = h1_ref[...]
    h1f = h1.reshape(_S * _N, _D)
    a1 = jnp.maximum(
        jnp.dot(h1f, w0_ref[...], preferred_element_type=f32)
        + jnp.dot(m2, aw0_ref[...], preferred_element_type=f32), 0.0)
    m1 = jnp.mean(h1, axis=1)
    h0 = h0_ref[...]
    a0 = jnp.maximum(
        jnp.dot(h0, w0_ref[...], preferred_element_type=f32)
        + jnp.dot(m1, aw0_ref[...], preferred_element_type=f32), 0.0)
    ma1 = jnp.mean(a1.reshape(_S, _N, _D), axis=1)
    w1 = w1_ref[...]
    aw1 = aw1_ref[...]
    hid = (jnp.dot(a0, w1[:_D], preferred_element_type=f32)
           + jnp.dot(h0, w1[_D:], preferred_element_type=f32)
           + jnp.dot(ma1, aw1[:_D], preferred_element_type=f32)
           + jnp.dot(m1, aw1[_D:], preferred_element_type=f32))
    fcw = fcw_ref[...]
    out_ref[...] = (jnp.dot(hid, fcw[:_D], preferred_element_type=f32)
                    + jnp.dot(h0, fcw[_D:], preferred_element_type=f32)
                    + fcb_ref[0])


def kernel(node_features_list_0, node_features_list_1, node_features_list_2,
           w0, aw0, w1, aw1, fc_w, fc_b):
    h0 = node_features_list_0
    h1 = node_features_list_1.reshape(_BATCH, _N, _D)
    h2 = node_features_list_2.reshape(_BATCH * _N, _N, _D)
    grid = (_BATCH // _S,)
    rep2 = lambda i: (0, 0)
    return pl.pallas_call(
        _fused,
        grid=grid,
        in_specs=[
            pl.BlockSpec((_S, _D), lambda i: (i, 0)),
            pl.BlockSpec((_S, _N, _D), lambda i: (i, 0, 0)),
            pl.BlockSpec((_S * _N, _N, _D), lambda i: (i, 0, 0)),
            pl.BlockSpec((_D, _D), rep2),
            pl.BlockSpec((_D, _D), rep2),
            pl.BlockSpec((2 * _D, _D), rep2),
            pl.BlockSpec((2 * _D, _D), rep2),
            pl.BlockSpec((2 * _D, 1), rep2),
            pl.BlockSpec(memory_space=pltpu.SMEM),
        ],
        out_specs=pl.BlockSpec((_S, 1), lambda i: (i, 0)),
        out_shape=jax.ShapeDtypeStruct((_BATCH, 1), jnp.float32),
        compiler_params=pltpu.CompilerParams(
            dimension_semantics=("arbitrary",),
        ),
    )(h0, h1, h2, w0, aw0, w1, aw1, fc_w, fc_b)
